# baseline (speedup 1.0000x reference)
import functools

import jax
import jax.numpy as jnp
from jax import lax
from jax.experimental import pallas as pl
from jax.experimental.pallas import tpu as pltpu
from jax.experimental.pallas import tpu_sc as plsc

RCUT_IM = 8.0
RCUT = 5.0
NMSG = 3
NRBF = 8
NEMB = 8

TILE = 1024


def _ceil_to(x, m):
    return (x + m - 1) // m * m


def _rbf(d, r_cut):
    width = r_cut / (NRBF - 1)
    centers = (lax.broadcasted_iota(jnp.int32, (1, NRBF), 1)
               .astype(jnp.float32) * width)
    beta = 1.0 / (width * width)
    dc = jnp.clip(d, 0.0, r_cut)
    fcut = 0.5 * (jnp.cos(jnp.pi * dc / r_cut) + 1.0)
    return jnp.exp(-beta * (d - centers) ** 2) * fcut


def _gates_body(sa_ref, ta_ref, sb_ref, tb_ref, w0_ref, w1_ref, w2_ref,
                o0_ref, o1_ref, o2_ref, *, T, E):
    i = pl.program_id(0)
    lane = lax.broadcasted_iota(jnp.int32, (T, 128), 1)
    sub = lane % NEMB
    row = lax.broadcasted_iota(jnp.int32, (T, 128), 0) + i * T
    e_idx = row * (128 // NEMB) + lane // NEMB
    valid = e_idx < E
    width = RCUT / (NRBF - 1)
    beta = 1.0 / (width * width)
    centers = sub.astype(jnp.float32) * width
    iso = sub == 0
    outs = (o0_ref, o1_ref, o2_ref)
    ws = (w0_ref, w1_ref, w2_ref)
    bf = jnp.bfloat16
    for m, (s_ref, t_ref) in enumerate(((sa_ref, ta_ref), (sb_ref, tb_ref))):
        dx = t_ref[0] - s_ref[0]
        sq = dx * dx
        v = sq + pltpu.roll(sq, 127, 1) + pltpu.roll(sq, 126, 1)
        keep = jnp.where(iso, v, 0.0)
        r = keep + pltpu.roll(keep, 1, 1)
        r = r + pltpu.roll(r, 2, 1)
        d2 = r + pltpu.roll(r, 4, 1)
        d = jnp.sqrt(d2)
        dc = jnp.clip(d, 0.0, RCUT)
        fcut = 0.5 * (jnp.cos(jnp.pi * dc / RCUT) + 1.0)
        rbf = jnp.exp(-beta * (d - centers) ** 2) * fcut
        rbf = jnp.where(valid, rbf, 0.0).astype(bf)
        for t in range(NMSG):
            outs[t][m] = jnp.dot(rbf, ws[t][...].astype(bf),
                                 preferred_element_type=jnp.float32)


def _gates_tc(posw, wbigs, Ep, E):
    SEG = Ep * NEMB // 128
    steps = SEG // TILE
    specs = [pl.BlockSpec((1, TILE, 128),
                          functools.partial(lambda k, i: (k, i, 0), k))
             for k in range(4)]
    specs += [pl.BlockSpec((128, 128), lambda i: (0, 0))] * NMSG
    out_spec = pl.BlockSpec((2, TILE, 128), lambda i: (0, i, 0))
    out_sh = jax.ShapeDtypeStruct((2, SEG, 128), jnp.float32)
    return pl.pallas_call(
        functools.partial(_gates_body, T=TILE, E=E),
        grid=(steps,),
        in_specs=specs,
        out_specs=[out_spec] * NMSG,
        out_shape=[out_sh] * NMSG,
    )(posw, posw, posw, posw, *wbigs)


def _update_body(agg_ref, h_ref, w_ref, u_ref, o_ref):
    w = w_ref[...]
    u = u_ref[...]
    for m in range(2):
        o_ref[m] = jnp.tanh(
            jnp.dot(agg_ref[m], w, preferred_element_type=jnp.float32)
            + jnp.dot(h_ref[m], u, preferred_element_type=jnp.float32))


def _update_tc(agg, h, w, u, NP):
    steps = NP // TILE
    spec = pl.BlockSpec((2, TILE, NEMB), lambda i: (0, i, 0))
    wspec = pl.BlockSpec((NEMB, NEMB), lambda i: (0, 0))
    return pl.pallas_call(
        _update_body,
        grid=(steps,),
        in_specs=[spec, spec, wspec, wspec],
        out_specs=spec,
        out_shape=jax.ShapeDtypeStruct((2, NP, NEMB), jnp.float32),
    )(agg, h, w, u)


def _pair_body(f_ref, w1_ref, w1r_ref, b1_ref, w2_ref,
               b2_ref, w3_ref, b3_ref, w4_ref, o_ref, acc_ref, *, T, E, nsteps):
    i = pl.program_id(0)

    @pl.when(i == 0)
    def _():
        acc_ref[...] = jnp.zeros_like(acc_ref)

    blk = f_ref[...]
    dxyz = blk[:, 88:91] - blk[:, 40:43]
    d2 = jnp.sum(dxyz * dxyz, axis=1, keepdims=True)
    d = jnp.sqrt(d2)
    rbf = _rbf(d, RCUT_IM)
    x = (jnp.dot(blk[:, 0:96], w1_ref[...], preferred_element_type=jnp.float32)
         + jnp.dot(rbf, w1r_ref[...][0:8], preferred_element_type=jnp.float32)
         + b1_ref[...])
    x = jax.nn.relu(x)
    x = jax.nn.relu(jnp.dot(x, w2_ref[...], preferred_element_type=jnp.float32)
                    + b2_ref[...])
    x = jax.nn.relu(jnp.dot(x, w3_ref[...], preferred_element_type=jnp.float32)
                    + b3_ref[...])
    valid = (i * T + lax.broadcasted_iota(jnp.int32, (T, 1), 0)) < E
    x = jnp.where(valid, x, 0.0)
    acc_ref[...] += jnp.sum(x.reshape(T // 8, 8, 128), axis=0)

    @pl.when(i == nsteps - 1)
    def _():
        tot = jnp.sum(acc_ref[...], axis=0, keepdims=True)
        o_ref[...] = jnp.dot(tot, w4_ref[...],
                             preferred_element_type=jnp.float32)


def _pair_tc(feats, Wp, Ep, E):
    w1a, w1b, w1r, b1, w2, b2, w3, b3, w4, b4 = Wp
    w1 = jnp.concatenate([w1a, w1b], axis=0)
    PT = 2048
    nsteps = Ep // PT
    fspec = pl.BlockSpec((PT, 128), lambda i: (i, 0))
    full = lambda shape: pl.BlockSpec(shape, lambda i: (0,) * len(shape))
    return pl.pallas_call(
        functools.partial(_pair_body, T=PT, E=E, nsteps=nsteps),
        grid=(nsteps,),
        in_specs=[fspec, full((96, 256)),
                  full((128, 256)), full((1, 256)), full((256, 128)),
                  full((1, 128)), full((128, 128)), full((1, 128)),
                  full((128, 128))],
        out_specs=full((1, 128)),
        out_shape=jax.ShapeDtypeStruct((1, 128), jnp.float32),
        scratch_shapes=[pltpu.VMEM((8, 128), jnp.float32)],
    )(feats, w1, w1r, b1, w2, b2, w3, b3, w4)


def _lig_body(f_ref, w1_ref, b1_ref, w2_ref, b2_ref, w3_ref, b3_ref, w4_ref,
              o_ref, acc_ref, *, T, N, nsteps):
    i = pl.program_id(0)

    @pl.when(i == 0)
    def _():
        acc_ref[...] = jnp.zeros_like(acc_ref)

    a = f_ref[...]
    bf = jnp.bfloat16
    x = jax.nn.relu(jnp.dot(a.astype(bf), w1_ref[...].astype(bf),
                            preferred_element_type=jnp.float32)
                    + b1_ref[...])
    x = jax.nn.relu(jnp.dot(x.astype(bf), w2_ref[...].astype(bf),
                            preferred_element_type=jnp.float32)
                    + b2_ref[...])
    x = jax.nn.relu(jnp.dot(x.astype(bf), w3_ref[...].astype(bf),
                            preferred_element_type=jnp.float32)
                    + b3_ref[...])
    valid = (i * T + lax.broadcasted_iota(jnp.int32, (T, 1), 0)) < N
    x = jnp.where(valid, x, 0.0)
    acc_ref[...] += jnp.sum(x.reshape(T // 8, 8, 128), axis=0)

    @pl.when(i == nsteps - 1)
    def _():
        tot = jnp.sum(acc_ref[...], axis=0, keepdims=True)
        o_ref[...] = jnp.dot(tot, w4_ref[...],
                             preferred_element_type=jnp.float32)


def _lig_tc(feat, Wl, NP, N):
    w1, b1, w2, b2, w3, b3, w4, b4 = Wl
    nsteps = NP // TILE
    fspec = pl.BlockSpec((TILE, 48), lambda i: (i, 0))
    full = lambda shape: pl.BlockSpec(shape, lambda i: (0,) * len(shape))
    return pl.pallas_call(
        functools.partial(_lig_body, T=TILE, N=N, nsteps=nsteps),
        grid=(nsteps,),
        in_specs=[fspec, full((48, 256)), full((1, 256)), full((256, 128)),
                  full((1, 128)), full((128, 128)), full((1, 128)),
                  full((128, 128))],
        out_specs=full((1, 128)),
        out_shape=jax.ShapeDtypeStruct((1, 128), jnp.float32),
        scratch_shapes=[pltpu.VMEM((8, 128), jnp.float32)],
    )(feat, w1, b1, w2, b2, w3, b3, w4)


NC, NS, LANES = 2, 16, 16
NW = NC * NS
CHUNK = 128
NBUF = 8


def _sc_gather(table, idx, D):
    RI = idx.shape[0]
    per_w_chunks = RI // (NW * CHUNK)
    groups = per_w_chunks // NBUF
    rem = per_w_chunks - groups * NBUF
    mesh = plsc.VectorSubcoreMesh(core_axis_name="c", subcore_axis_name="s")
    idx2d = idx.reshape(RI // CHUNK, CHUNK)

    @functools.partial(
        pl.kernel, mesh=mesh,
        out_type=jax.ShapeDtypeStruct((RI, D), jnp.float32),
        compiler_params=pltpu.CompilerParams(use_tc_tiling_on_sc=False),
        scratch_types=[
            pltpu.VMEM((NBUF, CHUNK), jnp.int32),
            pltpu.VMEM((NBUF * CHUNK, D), jnp.float32),
            pltpu.SemaphoreType.DMA,
        ],
    )
    def k(table_hbm, idx_hbm, out_hbm, idx_v, rows_v, gsem):
        wid = lax.axis_index("s") * NC + lax.axis_index("c")
        chunk_base = wid * per_w_chunks

        def grp(row0, nb):
            pltpu.sync_copy(idx_hbm.at[pl.ds(row0, nb)],
                            idx_v.at[pl.ds(0, nb)])
            copies = [
                pltpu.async_copy(
                    table_hbm.at[idx_v.at[b]],
                    rows_v.at[pl.ds(b * CHUNK, CHUNK)], gsem)
                for b in range(nb)
            ]
            for cp in copies:
                cp.wait()
            pltpu.sync_copy(rows_v.at[pl.ds(0, nb * CHUNK)],
                            out_hbm.at[pl.ds(row0 * CHUNK, nb * CHUNK)])

        def body(g, _):
            grp(chunk_base + g * NBUF, NBUF)
            return 0

        lax.fori_loop(0, groups, body, 0)
        if rem:
            grp(chunk_base + groups * NBUF, rem)

    return k(table, idx2d)


def _sc_scatter_add(vals, tgt2d, zc, NP, Ep):
    EpC = Ep // CHUNK
    cpt = EpC // NS
    npt = NP // NS
    groups = cpt // NBUF
    mesh = plsc.VectorSubcoreMesh(core_axis_name="c", subcore_axis_name="s")

    @functools.partial(
        pl.kernel, mesh=mesh,
        out_type=jax.ShapeDtypeStruct((2 * NP, NEMB), jnp.float32),
        compiler_params=pltpu.CompilerParams(use_tc_tiling_on_sc=False),
        scratch_types=[
            pltpu.VMEM((NBUF, CHUNK), jnp.int32),
            pltpu.VMEM((NBUF * CHUNK, NEMB), jnp.float32),
            pltpu.VMEM_SHARED((NP, NEMB), jnp.float32),
            pltpu.SemaphoreType.DMA,
        ],
    )
    def k(vals_hbm, tgt_hbm, zc_hbm, out_hbm, tidx_v, rows_v, acc, ssem):
        c = lax.axis_index("c")
        s = lax.axis_index("s")

        def zbody(i, _):
            pltpu.sync_copy(zc_hbm, acc.at[pl.ds(s * npt + i * CHUNK, CHUNK)])
            return 0
        lax.fori_loop(0, npt // CHUNK, zbody, 0)
        plsc.subcore_barrier()

        def group(row0, nb):
            pltpu.sync_copy(tgt_hbm.at[pl.ds(row0, nb)],
                            tidx_v.at[pl.ds(0, nb)])
            pltpu.sync_copy(vals_hbm.at[pl.ds(row0 * CHUNK, nb * CHUNK)],
                            rows_v.at[pl.ds(0, nb * CHUNK)])
            scs = [
                pltpu.async_copy(rows_v.at[pl.ds(b * CHUNK, CHUNK)],
                                 acc.at[tidx_v.at[b]], ssem, add=True)
                for b in range(nb)
            ]
            for cp in scs:
                cp.wait()

        def gbody(g, _):
            group((c * EpC + s * cpt) + g * NBUF, NBUF)
            return 0
        lax.fori_loop(0, groups, gbody, 0)

        plsc.subcore_barrier()
        pltpu.sync_copy(acc.at[pl.ds(s * npt, npt)],
                        out_hbm.at[pl.ds(c * NP + s * npt, npt)])

    return k(vals, tgt2d, zc)


def _sc_gather_pair(table, idx2d, Ep):
    EpC = Ep // CHUNK
    cpt = EpC // NS
    groups = cpt // NBUF
    mesh = plsc.VectorSubcoreMesh(core_axis_name="c", subcore_axis_name="s")

    @functools.partial(
        pl.kernel, mesh=mesh,
        out_type=jax.ShapeDtypeStruct((Ep, 128), jnp.float32),
        compiler_params=pltpu.CompilerParams(use_tc_tiling_on_sc=False),
        scratch_types=[
            pltpu.VMEM((NBUF, CHUNK), jnp.int32),
            pltpu.VMEM((NBUF * CHUNK, 48), jnp.float32),
            pltpu.SemaphoreType.DMA,
        ],
    )
    def k(table_hbm, idx_hbm, out_hbm, idx_v, rows_v, gsem):
        c = lax.axis_index("c")
        s = lax.axis_index("s")

        def grp(g, _):
            row0 = c * EpC + s * cpt + g * NBUF
            erow0 = (s * cpt + g * NBUF) * CHUNK
            pltpu.sync_copy(idx_hbm.at[pl.ds(row0, NBUF)], idx_v)
            copies = [
                pltpu.async_copy(table_hbm.at[idx_v.at[b]],
                                 rows_v.at[pl.ds(b * CHUNK, CHUNK)], gsem)
                for b in range(NBUF)
            ]
            for cp in copies:
                cp.wait()

            @pl.when(c == 0)
            def _():
                pltpu.sync_copy(
                    rows_v, out_hbm.at[pl.ds(erow0, NBUF * CHUNK),
                                       pl.ds(0, 48)])

            @pl.when(c == 1)
            def _():
                pltpu.sync_copy(
                    rows_v, out_hbm.at[pl.ds(erow0, NBUF * CHUNK),
                                       pl.ds(48, 48)])
            return 0

        lax.fori_loop(0, groups, grp, 0)

    return k(table, idx2d)


def _mul_body(a_ref, b_ref, o_ref):
    o_ref[...] = a_ref[...] * b_ref[...]


def _mul_tc(a, b):
    T = 2048
    steps = a.shape[0] // T
    spec = pl.BlockSpec((T, 128), lambda i: (i, 0))
    return pl.pallas_call(
        _mul_body,
        grid=(steps,),
        in_specs=[spec, spec],
        out_specs=spec,
        out_shape=jax.ShapeDtypeStruct(a.shape, jnp.float32),
    )(a, b)


def _table_body(ep_ref, h0_ref, h1_ref, h2_ref, h3_ref, r8_ref, o_ref):
    for m in range(2):
        o_ref[m, :, 0:8] = ep_ref[m]
        o_ref[m, :, 8:16] = h0_ref[m]
        o_ref[m, :, 16:24] = h1_ref[m]
        o_ref[m, :, 24:32] = h2_ref[m]
        o_ref[m, :, 32:40] = h3_ref[m]
        o_ref[m, :, 40:48] = r8_ref[m]


def _table_tc(epair, h0, h1, h2, h3, r8, NP):
    steps = NP // TILE
    s8 = pl.BlockSpec((2, TILE, NEMB), lambda i: (0, i, 0))
    return pl.pallas_call(
        _table_body,
        grid=(steps,),
        in_specs=[s8] * 6,
        out_specs=pl.BlockSpec((2, TILE, 48), lambda i: (0, i, 0)),
        out_shape=jax.ShapeDtypeStruct((2, NP, 48), jnp.float32),
    )(epair, h0, h1, h2, h3, r8)


def _prep_pair_weights(params):
    W = params['pair_W']
    b = params['pair_b']
    w1 = W[0]
    w1a = jnp.zeros((48, 256), jnp.float32).at[:40].set(w1[:40])
    w1b = jnp.zeros((48, 256), jnp.float32).at[:40].set(w1[40:80])
    w1r = jnp.zeros((128, 256), jnp.float32).at[:8].set(w1[80:88])
    b1 = b[0].reshape(1, 256)
    w2 = W[1]
    b2 = b[1].reshape(1, 128)
    w3 = jnp.zeros((128, 128), jnp.float32).at[:, :64].set(W[2])
    b3 = jnp.zeros((1, 128), jnp.float32).at[:, :64].set(b[2])
    w4 = jnp.zeros((128, 128), jnp.float32).at[:64, 0:1].set(W[3])
    return (w1a, w1b, w1r, b1, w2, b2, w3, b3, w4, b[3][0])


def _prep_lig_weights(params):
    W = params['lig_W']
    b = params['lig_b']
    w1 = jnp.zeros((48, 256), jnp.float32).at[:40].set(W[0])
    b1 = b[0].reshape(1, 256)
    w2 = W[1]
    b2 = b[1].reshape(1, 128)
    w3 = jnp.zeros((128, 128), jnp.float32).at[:, :64].set(W[2])
    b3 = jnp.zeros((1, 128), jnp.float32).at[:, :64].set(b[2])
    w4 = jnp.zeros((128, 128), jnp.float32).at[:64, 0:1].set(W[3])
    return (w1, b1, w2, b2, w3, b3, w4, b[3][0])


def kernel(ZA, RA, ZB, RB, e_ABsr_source, e_ABsr_target, dimer_ind,
           e_AA_source, e_AA_target, e_BB_source, e_BB_target,
           monomerA_ind, monomerB_ind, total_charge_A, total_charge_B, params):
    ZA = ZA[0].astype(jnp.int32)
    ZB = ZB[0].astype(jnp.int32)
    RA = RA[0]
    RB = RB[0]
    sAB = e_ABsr_source[0].astype(jnp.int32)
    tAB = e_ABsr_target[0].astype(jnp.int32)
    sAA = e_AA_source[0].astype(jnp.int32)
    tAA = e_AA_target[0].astype(jnp.int32)
    sBB = e_BB_source[0].astype(jnp.int32)
    tBB = e_BB_target[0].astype(jnp.int32)

    natom = ZA.shape[0]
    E = sAB.shape[0]
    NP = _ceil_to(natom, 2048)
    Ep = _ceil_to(E, NS * CHUNK * NBUF)

    padi = lambda x, n: jnp.pad(x, (0, n - x.shape[0]))

    sAA_p = padi(sAA, Ep)
    tAA_p = padi(tAA, Ep)
    sBB_p = padi(sBB, Ep)
    tBB_p = padi(tBB, Ep)
    sAB_p = padi(sAB, Ep)
    tAB_p = padi(tAB, Ep)
    ZA_p = padi(ZA, NP)
    ZB_p = padi(ZB, NP)

    R8 = jnp.zeros((2, NP, 8), jnp.float32)
    R8 = R8.at[0, :natom, :3].set(RA).at[1, :natom, :3].set(RB)
    R8_flat = R8.reshape(2 * NP, 8)

    idxZ = padi(jnp.concatenate([ZA_p, ZB_p]),
                _ceil_to(2 * NP, NW * CHUNK * NBUF))
    epair = _sc_gather(params['embed_pair'], idxZ,
                       NEMB)[:2 * NP].reshape(2, NP, NEMB)
    h = _sc_gather(params['embed_atom'], idxZ,
                   NEMB)[:2 * NP].reshape(2, NP, NEMB)

    pos_idx = jnp.concatenate([sAA_p, tAA_p, sBB_p + NP, tBB_p + NP])
    posG = _sc_gather(R8_flat, pos_idx, 8)
    posw = posG.reshape(4, Ep * NEMB // 128, 128)
    wbigs = [jnp.kron(jnp.eye(128 // NEMB, dtype=jnp.float32),
                      params['W_rbf'][t]) for t in range(NMSG)]
    gates = _gates_tc(posw, wbigs, Ep, E)

    src_flat = jnp.stack([sAA_p, sBB_p + NP]).reshape(2 * Ep)
    tgt2d = jnp.stack([tAA_p, tBB_p]).reshape(2 * Ep // CHUNK, CHUNK)
    zc = jnp.zeros((CHUNK, NEMB), jnp.float32)
    WROWS = 2 * Ep * NEMB // 128
    hs = [h]
    for t in range(NMSG):
        rows = _sc_gather(hs[-1].reshape(2 * NP, NEMB), src_flat, NEMB)
        msg = _mul_tc(rows.reshape(WROWS, 128), gates[t].reshape(WROWS, 128))
        agg = _sc_scatter_add(msg.reshape(2 * Ep, NEMB), tgt2d, zc, NP,
                              Ep).reshape(2, NP, NEMB)
        hs.append(_update_tc(agg, hs[-1], params['W_msg'][t],
                             params['U_msg'][t], NP))

    table = _table_tc(epair, hs[0], hs[1], hs[2], hs[3], R8, NP)
    table_flat = table.reshape(2 * NP, 48)

    pair_idx2d = jnp.concatenate([sAB_p, tAB_p + NP]).reshape(
        2 * Ep // CHUNK, CHUNK)
    feats = _sc_gather_pair(table_flat, pair_idx2d, Ep)
    Wp = _prep_pair_weights(params)
    pair_out = _pair_tc(feats, Wp, Ep, E)
    pair_pred = (pair_out[0, 0] + E * Wp[-1]) * 3e-06

    Wl = _prep_lig_weights(params)
    lig_out = _lig_tc(table[0], Wl, NP, natom)
    lig_pred = (lig_out[0, 0] + natom * Wl[-1]) * 1e-04

    dG = pair_pred - lig_pred + params['shift']
    return dG.reshape(1)


# speedup vs baseline: 1.0587x; 1.0587x over previous
import functools

import jax
import jax.numpy as jnp
from jax import lax
from jax.experimental import pallas as pl
from jax.experimental.pallas import tpu as pltpu
from jax.experimental.pallas import tpu_sc as plsc

RCUT_IM = 8.0
RCUT = 5.0
NMSG = 3
NRBF = 8
NEMB = 8

TILE = 1024


def _ceil_to(x, m):
    return (x + m - 1) // m * m


def _rbf(d, r_cut):
    width = r_cut / (NRBF - 1)
    centers = (lax.broadcasted_iota(jnp.int32, (1, NRBF), 1)
               .astype(jnp.float32) * width)
    beta = 1.0 / (width * width)
    dc = jnp.clip(d, 0.0, r_cut)
    fcut = 0.5 * (jnp.cos(jnp.pi * dc / r_cut) + 1.0)
    return jnp.exp(-beta * (d - centers) ** 2) * fcut


def _gates_body(sa_ref, ta_ref, sb_ref, tb_ref, w0_ref, w1_ref, w2_ref,
                o0_ref, o1_ref, o2_ref, *, T, E):
    i = pl.program_id(0)
    lane = lax.broadcasted_iota(jnp.int32, (T, 128), 1)
    sub = lane % NEMB
    row = lax.broadcasted_iota(jnp.int32, (T, 128), 0) + i * T
    e_idx = row * (128 // NEMB) + lane // NEMB
    valid = e_idx < E
    width = RCUT / (NRBF - 1)
    beta = 1.0 / (width * width)
    centers = sub.astype(jnp.float32) * width
    iso = sub == 0
    outs = (o0_ref, o1_ref, o2_ref)
    ws = (w0_ref, w1_ref, w2_ref)
    bf = jnp.bfloat16
    for m, (s_ref, t_ref) in enumerate(((sa_ref, ta_ref), (sb_ref, tb_ref))):
        dx = t_ref[0] - s_ref[0]
        sq = dx * dx
        v = sq + pltpu.roll(sq, 127, 1) + pltpu.roll(sq, 126, 1)
        keep = jnp.where(iso, v, 0.0)
        r = keep + pltpu.roll(keep, 1, 1)
        r = r + pltpu.roll(r, 2, 1)
        d2 = r + pltpu.roll(r, 4, 1)
        d = jnp.sqrt(d2)
        dc = jnp.clip(d, 0.0, RCUT)
        fcut = 0.5 * (jnp.cos(jnp.pi * dc / RCUT) + 1.0)
        rbf = jnp.exp(-beta * (d - centers) ** 2) * fcut
        rbf = jnp.where(valid, rbf, 0.0).astype(bf)
        for t in range(NMSG):
            outs[t][m] = jnp.dot(rbf, ws[t][...].astype(bf),
                                 preferred_element_type=jnp.float32)


def _gates_tc(posw, wbigs, Ep, E):
    SEG = Ep * NEMB // 128
    steps = SEG // TILE
    specs = [pl.BlockSpec((1, TILE, 128),
                          functools.partial(lambda k, i: (k, i, 0), k))
             for k in range(4)]
    specs += [pl.BlockSpec((128, 128), lambda i: (0, 0))] * NMSG
    out_spec = pl.BlockSpec((2, TILE, 128), lambda i: (0, i, 0))
    out_sh = jax.ShapeDtypeStruct((2, SEG, 128), jnp.float32)
    return pl.pallas_call(
        functools.partial(_gates_body, T=TILE, E=E),
        grid=(steps,),
        in_specs=specs,
        out_specs=[out_spec] * NMSG,
        out_shape=[out_sh] * NMSG,
    )(posw, posw, posw, posw, *wbigs)


def _update_body(agg_ref, h_ref, w_ref, u_ref, o_ref):
    w = w_ref[...]
    u = u_ref[...]
    for m in range(2):
        o_ref[m] = jnp.tanh(
            jnp.dot(agg_ref[m], w, preferred_element_type=jnp.float32)
            + jnp.dot(h_ref[m], u, preferred_element_type=jnp.float32))


def _update_tc(agg, h, w, u, NP):
    steps = NP // TILE
    spec = pl.BlockSpec((2, TILE, NEMB), lambda i: (0, i, 0))
    wspec = pl.BlockSpec((NEMB, NEMB), lambda i: (0, 0))
    return pl.pallas_call(
        _update_body,
        grid=(steps,),
        in_specs=[spec, spec, wspec, wspec],
        out_specs=spec,
        out_shape=jax.ShapeDtypeStruct((2, NP, NEMB), jnp.float32),
    )(agg, h, w, u)


def _pair_body(f_ref, w1_ref, w1r_ref, b1_ref, w2_ref,
               b2_ref, w3_ref, b3_ref, w4_ref, o_ref, acc_ref, *, T, E, nsteps):
    i = pl.program_id(0)

    @pl.when(i == 0)
    def _():
        acc_ref[...] = jnp.zeros_like(acc_ref)

    blk = f_ref[...]
    dxyz = blk[:, 88:91] - blk[:, 40:43]
    d2 = jnp.sum(dxyz * dxyz, axis=1, keepdims=True)
    d = jnp.sqrt(d2)
    rbf = _rbf(d, RCUT_IM)
    x = (jnp.dot(blk[:, 0:96], w1_ref[...], preferred_element_type=jnp.float32)
         + jnp.dot(rbf, w1r_ref[...][0:8], preferred_element_type=jnp.float32)
         + b1_ref[...])
    x = jax.nn.relu(x)
    x = jax.nn.relu(jnp.dot(x, w2_ref[...], preferred_element_type=jnp.float32)
                    + b2_ref[...])
    x = jax.nn.relu(jnp.dot(x, w3_ref[...], preferred_element_type=jnp.float32)
                    + b3_ref[...])
    valid = (i * T + lax.broadcasted_iota(jnp.int32, (T, 1), 0)) < E
    x = jnp.where(valid, x, 0.0)
    acc_ref[...] += jnp.sum(x.reshape(T // 8, 8, 128), axis=0)

    @pl.when(i == nsteps - 1)
    def _():
        tot = jnp.sum(acc_ref[...], axis=0, keepdims=True)
        o_ref[...] = jnp.dot(tot, w4_ref[...],
                             preferred_element_type=jnp.float32)


def _pair_tc(feats, Wp, Ep, E):
    w1a, w1b, w1r, b1, w2, b2, w3, b3, w4, b4 = Wp
    w1 = jnp.concatenate([w1a, w1b], axis=0)
    PT = 2048
    nsteps = Ep // PT
    fspec = pl.BlockSpec((PT, 128), lambda i: (i, 0))
    full = lambda shape: pl.BlockSpec(shape, lambda i: (0,) * len(shape))
    return pl.pallas_call(
        functools.partial(_pair_body, T=PT, E=E, nsteps=nsteps),
        grid=(nsteps,),
        in_specs=[fspec, full((96, 256)),
                  full((128, 256)), full((1, 256)), full((256, 128)),
                  full((1, 128)), full((128, 128)), full((1, 128)),
                  full((128, 128))],
        out_specs=full((1, 128)),
        out_shape=jax.ShapeDtypeStruct((1, 128), jnp.float32),
        scratch_shapes=[pltpu.VMEM((8, 128), jnp.float32)],
    )(feats, w1, w1r, b1, w2, b2, w3, b3, w4)


def _lig_body(f_ref, w1_ref, b1_ref, w2_ref, b2_ref, w3_ref, b3_ref, w4_ref,
              o_ref, acc_ref, *, T, N, nsteps):
    i = pl.program_id(0)

    @pl.when(i == 0)
    def _():
        acc_ref[...] = jnp.zeros_like(acc_ref)

    a = f_ref[...]
    bf = jnp.bfloat16
    x = jax.nn.relu(jnp.dot(a.astype(bf), w1_ref[...].astype(bf),
                            preferred_element_type=jnp.float32)
                    + b1_ref[...])
    x = jax.nn.relu(jnp.dot(x.astype(bf), w2_ref[...].astype(bf),
                            preferred_element_type=jnp.float32)
                    + b2_ref[...])
    x = jax.nn.relu(jnp.dot(x.astype(bf), w3_ref[...].astype(bf),
                            preferred_element_type=jnp.float32)
                    + b3_ref[...])
    valid = (i * T + lax.broadcasted_iota(jnp.int32, (T, 1), 0)) < N
    x = jnp.where(valid, x, 0.0)
    acc_ref[...] += jnp.sum(x.reshape(T // 8, 8, 128), axis=0)

    @pl.when(i == nsteps - 1)
    def _():
        tot = jnp.sum(acc_ref[...], axis=0, keepdims=True)
        o_ref[...] = jnp.dot(tot, w4_ref[...],
                             preferred_element_type=jnp.float32)


def _lig_tc(feat, Wl, NP, N):
    w1, b1, w2, b2, w3, b3, w4, b4 = Wl
    nsteps = NP // TILE
    fspec = pl.BlockSpec((TILE, 48), lambda i: (i, 0))
    full = lambda shape: pl.BlockSpec(shape, lambda i: (0,) * len(shape))
    return pl.pallas_call(
        functools.partial(_lig_body, T=TILE, N=N, nsteps=nsteps),
        grid=(nsteps,),
        in_specs=[fspec, full((48, 256)), full((1, 256)), full((256, 128)),
                  full((1, 128)), full((128, 128)), full((1, 128)),
                  full((128, 128))],
        out_specs=full((1, 128)),
        out_shape=jax.ShapeDtypeStruct((1, 128), jnp.float32),
        scratch_shapes=[pltpu.VMEM((8, 128), jnp.float32)],
    )(feat, w1, b1, w2, b2, w3, b3, w4)


NC, NS, LANES = 2, 16, 16
NW = NC * NS
CHUNK = 128
NBUF = 8


def _sc_gather(table, idx, D):
    RI = idx.shape[0]
    per_w_chunks = RI // (NW * CHUNK)
    groups = per_w_chunks // NBUF
    rem = per_w_chunks - groups * NBUF
    mesh = plsc.VectorSubcoreMesh(core_axis_name="c", subcore_axis_name="s")
    idx2d = idx.reshape(RI // CHUNK, CHUNK)

    @functools.partial(
        pl.kernel, mesh=mesh,
        out_type=jax.ShapeDtypeStruct((RI, D), jnp.float32),
        compiler_params=pltpu.CompilerParams(use_tc_tiling_on_sc=False),
        scratch_types=[
            pltpu.VMEM((NBUF, CHUNK), jnp.int32),
            pltpu.VMEM((NBUF * CHUNK, D), jnp.float32),
            pltpu.SemaphoreType.DMA,
        ],
    )
    def k(table_hbm, idx_hbm, out_hbm, idx_v, rows_v, gsem):
        wid = lax.axis_index("s") * NC + lax.axis_index("c")
        chunk_base = wid * per_w_chunks

        def grp(row0, nb):
            pltpu.sync_copy(idx_hbm.at[pl.ds(row0, nb)],
                            idx_v.at[pl.ds(0, nb)])
            copies = [
                pltpu.async_copy(
                    table_hbm.at[idx_v.at[b]],
                    rows_v.at[pl.ds(b * CHUNK, CHUNK)], gsem)
                for b in range(nb)
            ]
            for cp in copies:
                cp.wait()
            pltpu.sync_copy(rows_v.at[pl.ds(0, nb * CHUNK)],
                            out_hbm.at[pl.ds(row0 * CHUNK, nb * CHUNK)])

        def body(g, _):
            grp(chunk_base + g * NBUF, NBUF)
            return 0

        lax.fori_loop(0, groups, body, 0)
        if rem:
            grp(chunk_base + groups * NBUF, rem)

    return k(table, idx2d)


def _sc_scatter_add(vals, tgt2d, zc, NP, Ep):
    EpC = Ep // CHUNK
    cpt = EpC // NS
    npt = NP // NS
    groups = cpt // NBUF
    mesh = plsc.VectorSubcoreMesh(core_axis_name="c", subcore_axis_name="s")

    @functools.partial(
        pl.kernel, mesh=mesh,
        out_type=jax.ShapeDtypeStruct((2 * NP, NEMB), jnp.float32),
        compiler_params=pltpu.CompilerParams(use_tc_tiling_on_sc=False),
        scratch_types=[
            pltpu.VMEM((NBUF, CHUNK), jnp.int32),
            pltpu.VMEM((NBUF * CHUNK, NEMB), jnp.float32),
            pltpu.VMEM_SHARED((NP, NEMB), jnp.float32),
            pltpu.SemaphoreType.DMA,
        ],
    )
    def k(vals_hbm, tgt_hbm, zc_hbm, out_hbm, tidx_v, rows_v, acc, ssem):
        c = lax.axis_index("c")
        s = lax.axis_index("s")

        def zbody(i, _):
            pltpu.sync_copy(zc_hbm, acc.at[pl.ds(s * npt + i * CHUNK, CHUNK)])
            return 0
        lax.fori_loop(0, npt // CHUNK, zbody, 0)
        plsc.subcore_barrier()

        def group(row0, nb):
            pltpu.sync_copy(tgt_hbm.at[pl.ds(row0, nb)],
                            tidx_v.at[pl.ds(0, nb)])
            pltpu.sync_copy(vals_hbm.at[pl.ds(row0 * CHUNK, nb * CHUNK)],
                            rows_v.at[pl.ds(0, nb * CHUNK)])
            scs = [
                pltpu.async_copy(rows_v.at[pl.ds(b * CHUNK, CHUNK)],
                                 acc.at[tidx_v.at[b]], ssem, add=True)
                for b in range(nb)
            ]
            for cp in scs:
                cp.wait()

        def gbody(g, _):
            group((c * EpC + s * cpt) + g * NBUF, NBUF)
            return 0
        lax.fori_loop(0, groups, gbody, 0)

        plsc.subcore_barrier()
        pltpu.sync_copy(acc.at[pl.ds(s * npt, npt)],
                        out_hbm.at[pl.ds(c * NP + s * npt, npt)])

    return k(vals, tgt2d, zc)


def _sc_gather_pair(table, idx2d, Ep):
    EpC = Ep // CHUNK
    cpt = EpC // NS
    groups = cpt // NBUF
    mesh = plsc.VectorSubcoreMesh(core_axis_name="c", subcore_axis_name="s")

    @functools.partial(
        pl.kernel, mesh=mesh,
        out_type=jax.ShapeDtypeStruct((Ep, 128), jnp.float32),
        compiler_params=pltpu.CompilerParams(use_tc_tiling_on_sc=False),
        scratch_types=[
            pltpu.VMEM((NBUF, CHUNK), jnp.int32),
            pltpu.VMEM((NBUF * CHUNK, 48), jnp.float32),
            pltpu.SemaphoreType.DMA,
        ],
    )
    def k(table_hbm, idx_hbm, out_hbm, idx_v, rows_v, gsem):
        c = lax.axis_index("c")
        s = lax.axis_index("s")

        def grp(g, _):
            row0 = c * EpC + s * cpt + g * NBUF
            erow0 = (s * cpt + g * NBUF) * CHUNK
            pltpu.sync_copy(idx_hbm.at[pl.ds(row0, NBUF)], idx_v)
            copies = [
                pltpu.async_copy(table_hbm.at[idx_v.at[b]],
                                 rows_v.at[pl.ds(b * CHUNK, CHUNK)], gsem)
                for b in range(NBUF)
            ]
            for cp in copies:
                cp.wait()

            @pl.when(c == 0)
            def _():
                pltpu.sync_copy(
                    rows_v, out_hbm.at[pl.ds(erow0, NBUF * CHUNK),
                                       pl.ds(0, 48)])

            @pl.when(c == 1)
            def _():
                pltpu.sync_copy(
                    rows_v, out_hbm.at[pl.ds(erow0, NBUF * CHUNK),
                                       pl.ds(48, 48)])
            return 0

        lax.fori_loop(0, groups, grp, 0)

    return k(table, idx2d)


def _mul_body(a_ref, b_ref, o_ref):
    o_ref[...] = a_ref[...] * b_ref[...]


def _mul_tc(a, b):
    T = 2048
    steps = a.shape[0] // T
    spec = pl.BlockSpec((T, 128), lambda i: (i, 0))
    return pl.pallas_call(
        _mul_body,
        grid=(steps,),
        in_specs=[spec, spec],
        out_specs=spec,
        out_shape=jax.ShapeDtypeStruct(a.shape, jnp.float32),
    )(a, b)


def _table_body(emb_ref, h1_ref, h2_ref, h3_ref, r8_ref, o_ref):
    for m in range(2):
        o_ref[m, :, 0:16] = emb_ref[m]
        o_ref[m, :, 16:24] = h1_ref[m]
        o_ref[m, :, 24:32] = h2_ref[m]
        o_ref[m, :, 32:40] = h3_ref[m]
        o_ref[m, :, 40:48] = r8_ref[m]


def _table_tc(emb, h1, h2, h3, r8, NP):
    steps = NP // TILE
    s16 = pl.BlockSpec((2, TILE, 16), lambda i: (0, i, 0))
    s8 = pl.BlockSpec((2, TILE, NEMB), lambda i: (0, i, 0))
    return pl.pallas_call(
        _table_body,
        grid=(steps,),
        in_specs=[s16, s8, s8, s8, s8],
        out_specs=pl.BlockSpec((2, TILE, 48), lambda i: (0, i, 0)),
        out_shape=jax.ShapeDtypeStruct((2, NP, 48), jnp.float32),
    )(emb, h1, h2, h3, r8)


def _prep_pair_weights(params):
    W = params['pair_W']
    b = params['pair_b']
    w1 = W[0]
    w1a = jnp.zeros((48, 256), jnp.float32).at[:40].set(w1[:40])
    w1b = jnp.zeros((48, 256), jnp.float32).at[:40].set(w1[40:80])
    w1r = jnp.zeros((128, 256), jnp.float32).at[:8].set(w1[80:88])
    b1 = b[0].reshape(1, 256)
    w2 = W[1]
    b2 = b[1].reshape(1, 128)
    w3 = jnp.zeros((128, 128), jnp.float32).at[:, :64].set(W[2])
    b3 = jnp.zeros((1, 128), jnp.float32).at[:, :64].set(b[2])
    w4 = jnp.zeros((128, 128), jnp.float32).at[:64, 0:1].set(W[3])
    return (w1a, w1b, w1r, b1, w2, b2, w3, b3, w4, b[3][0])


def _prep_lig_weights(params):
    W = params['lig_W']
    b = params['lig_b']
    w1 = jnp.zeros((48, 256), jnp.float32).at[:40].set(W[0])
    b1 = b[0].reshape(1, 256)
    w2 = W[1]
    b2 = b[1].reshape(1, 128)
    w3 = jnp.zeros((128, 128), jnp.float32).at[:, :64].set(W[2])
    b3 = jnp.zeros((1, 128), jnp.float32).at[:, :64].set(b[2])
    w4 = jnp.zeros((128, 128), jnp.float32).at[:64, 0:1].set(W[3])
    return (w1, b1, w2, b2, w3, b3, w4, b[3][0])


def kernel(ZA, RA, ZB, RB, e_ABsr_source, e_ABsr_target, dimer_ind,
           e_AA_source, e_AA_target, e_BB_source, e_BB_target,
           monomerA_ind, monomerB_ind, total_charge_A, total_charge_B, params):
    ZA = ZA[0].astype(jnp.int32)
    ZB = ZB[0].astype(jnp.int32)
    RA = RA[0]
    RB = RB[0]
    sAB = e_ABsr_source[0].astype(jnp.int32)
    tAB = e_ABsr_target[0].astype(jnp.int32)
    sAA = e_AA_source[0].astype(jnp.int32)
    tAA = e_AA_target[0].astype(jnp.int32)
    sBB = e_BB_source[0].astype(jnp.int32)
    tBB = e_BB_target[0].astype(jnp.int32)

    natom = ZA.shape[0]
    E = sAB.shape[0]
    NP = _ceil_to(natom, 2048)
    Ep = _ceil_to(E, NS * CHUNK * NBUF)

    padi = lambda x, n: jnp.pad(x, (0, n - x.shape[0]))

    sAA_p = padi(sAA, Ep)
    tAA_p = padi(tAA, Ep)
    sBB_p = padi(sBB, Ep)
    tBB_p = padi(tBB, Ep)
    sAB_p = padi(sAB, Ep)
    tAB_p = padi(tAB, Ep)
    ZA_p = padi(ZA, NP)
    ZB_p = padi(ZB, NP)

    R8 = jnp.zeros((2, NP, 8), jnp.float32)
    R8 = R8.at[0, :natom, :3].set(RA).at[1, :natom, :3].set(RB)
    R8_flat = R8.reshape(2 * NP, 8)

    ET = jnp.concatenate([params['embed_pair'], params['embed_atom']], axis=1)
    idxZ = padi(jnp.concatenate([ZA_p, ZB_p]),
                _ceil_to(2 * NP, NW * CHUNK * NBUF))
    emb = _sc_gather(ET, idxZ, 2 * NEMB)[:2 * NP].reshape(2, NP, 2 * NEMB)
    h = emb[:, :, NEMB:]

    pos_idx = jnp.concatenate([sAA_p, tAA_p, sBB_p + NP, tBB_p + NP])
    posG = _sc_gather(R8_flat, pos_idx, 8)
    posw = posG.reshape(4, Ep * NEMB // 128, 128)
    wbigs = [jnp.kron(jnp.eye(128 // NEMB, dtype=jnp.float32),
                      params['W_rbf'][t]) for t in range(NMSG)]
    gates = _gates_tc(posw, wbigs, Ep, E)

    src_flat = jnp.stack([sAA_p, sBB_p + NP]).reshape(2 * Ep)
    tgt2d = jnp.stack([tAA_p, tBB_p]).reshape(2 * Ep // CHUNK, CHUNK)
    zc = jnp.zeros((CHUNK, NEMB), jnp.float32)
    WROWS = 2 * Ep * NEMB // 128
    hs = [h]
    for t in range(NMSG):
        rows = _sc_gather(hs[-1].reshape(2 * NP, NEMB), src_flat, NEMB)
        msg = _mul_tc(rows.reshape(WROWS, 128), gates[t].reshape(WROWS, 128))
        agg = _sc_scatter_add(msg.reshape(2 * Ep, NEMB), tgt2d, zc, NP,
                              Ep).reshape(2, NP, NEMB)
        hs.append(_update_tc(agg, hs[-1], params['W_msg'][t],
                             params['U_msg'][t], NP))

    table = _table_tc(emb, hs[1], hs[2], hs[3], R8, NP)
    table_flat = table.reshape(2 * NP, 48)

    pair_idx2d = jnp.concatenate([sAB_p, tAB_p + NP]).reshape(
        2 * Ep // CHUNK, CHUNK)
    feats = _sc_gather_pair(table_flat, pair_idx2d, Ep)
    Wp = _prep_pair_weights(params)
    pair_out = _pair_tc(feats, Wp, Ep, E)
    pair_pred = (pair_out[0, 0] + E * Wp[-1]) * 3e-06

    Wl = _prep_lig_weights(params)
    lig_out = _lig_tc(table[0], Wl, NP, natom)
    lig_pred = (lig_out[0, 0] + natom * Wl[-1]) * 1e-04

    dG = pair_pred - lig_pred + params['shift']
    return dG.reshape(1)
